# baseline (speedup 1.0000x reference)
import jax
import jax.numpy as jnp
from jax.experimental import pallas as pl
from jax.experimental.pallas import tpu as pltpu

_N = 100000
_K = 64
_BLK = 2000
_NB = _N // _BLK


def _body(w0_ref, x3, w3, xcol, uV, bV,
          out_ref, acc_u, acc_ts, acc_sq, acc_b):
    i = pl.program_id(0)

    @pl.when(i == 0)
    def _init():
        acc_u[...] = jnp.zeros_like(acc_u)
        acc_ts[...] = jnp.zeros_like(acc_ts)
        acc_sq[...] = jnp.zeros_like(acc_sq)
        acc_b[...] = jnp.zeros_like(acc_b)

    xu_v = x3[i, :, :]
    xt_v = x3[_NB + i, :, :]
    xb_v = x3[2 * _NB + i, :, :]
    u_blk = uV[...]
    b_blk = bV[...]
    maskw = (xb_v == 1.0).astype(jnp.float32)
    mask_col = (xcol[i, :, :] == 1.0).astype(jnp.float32)

    acc_u[...] += jnp.dot(xu_v, u_blk, preferred_element_type=jnp.float32)
    lhs = jnp.concatenate([xt_v, maskw], axis=0)
    acc_ts[...] += jnp.dot(lhs, b_blk, preferred_element_type=jnp.float32)
    sq = jnp.sum((b_blk * b_blk) * mask_col)
    wsum = jnp.sum(xu_v * w3[i, :, :] + xt_v * w3[_NB + i, :, :]
                   + xb_v * w3[2 * _NB + i, :, :])
    acc_sq[...] += jnp.reshape(sq, (1, 1))
    acc_b[...] += jnp.reshape(wsum, (1, 1))

    @pl.when(i == _NB - 1)
    def _fin():
        u = acc_u[...]
        t = acc_ts[0:1, :]
        s = acc_ts[1:2, :]
        u_t = jnp.sum(u * t)
        t_b = jnp.sum(t * s)
        u_b = jnp.sum(u * s)
        bs = 0.5 * (jnp.sum(s * s) - acc_sq[0, 0])
        y = w0_ref[0, 0] + acc_b[0, 0] + u_t + t_b + bs + u_b
        out_ref[...] = jnp.reshape(y, (1, 1))


_VSPEC = pl.BlockSpec((_BLK, _K), lambda i: (i, 0))


@jax.jit
def _fm(x, w_0, w_bias, u_V, b_V):
    x3 = x.reshape(3 * _NB, 1, _BLK)
    w3 = w_bias.reshape(3 * _NB, 1, _BLK)
    xcol = x[2 * _N:].reshape(_NB, _BLK, 1)
    w0 = w_0.reshape(1, 1)
    return pl.pallas_call(
        _body,
        grid=(_NB,),
        in_specs=[
            pl.BlockSpec((1, 1), lambda i: (0, 0)),
            pl.BlockSpec((3 * _NB, 1, _BLK), lambda i: (0, 0, 0)),
            pl.BlockSpec((3 * _NB, 1, _BLK), lambda i: (0, 0, 0)),
            pl.BlockSpec((_NB, _BLK, 1), lambda i: (0, 0, 0)),
            _VSPEC, _VSPEC,
        ],
        out_specs=pl.BlockSpec((1, 1), lambda i: (0, 0)),
        out_shape=jax.ShapeDtypeStruct((1, 1), jnp.float32),
        scratch_shapes=[
            pltpu.VMEM((1, _K), jnp.float32),
            pltpu.VMEM((2, _K), jnp.float32),
            pltpu.VMEM((1, 1), jnp.float32),
            pltpu.VMEM((1, 1), jnp.float32),
        ],
    )(w0, x3, w3, xcol, u_V, b_V)


def kernel(x, delta, pmi, w_0, w_bias, u_V, b_V):
    return _fm(x, w_0, w_bias, u_V, b_V)


# speedup vs baseline: 1.2975x; 1.2975x over previous
import jax
import jax.numpy as jnp
from jax.experimental import pallas as pl
from jax.experimental.pallas import tpu as pltpu

_N = 100000
_K = 64
_BLK = 10000
_NB = _N // _BLK
_SUB = _BLK // 8


def _body(w0_ref, x3, w3, uV, bV,
          out_ref, acc_u, acc_ts, acc_sq, acc_b):
    i = pl.program_id(0)

    @pl.when(i == 0)
    def _init():
        acc_u[...] = jnp.zeros_like(acc_u)
        acc_ts[...] = jnp.zeros_like(acc_ts)
        acc_sq[...] = jnp.zeros_like(acc_sq)
        acc_b[...] = jnp.zeros_like(acc_b)

    xu8 = x3[i, :, :]
    xt8 = x3[_NB + i, :, :]
    xb8 = x3[2 * _NB + i, :, :]
    m8 = (xb8 == 1.0).astype(jnp.float32)

    du = jnp.zeros((1, _K), jnp.float32)
    dts = jnp.zeros((2, _K), jnp.float32)
    dsq = jnp.zeros((1, _K), jnp.float32)
    for s in range(8):
        u_sb = uV[0, s, :, :]
        b_sb = bV[0, s, :, :]
        du += jnp.dot(xu8[s:s + 1, :], u_sb,
                      preferred_element_type=jnp.float32)
        lhs = jnp.concatenate([xt8[s:s + 1, :], m8[s:s + 1, :]], axis=0)
        dts += jnp.dot(lhs, b_sb, preferred_element_type=jnp.float32)
        dsq += jnp.dot(m8[s:s + 1, :], b_sb * b_sb,
                       preferred_element_type=jnp.float32)
    acc_u[...] += du
    acc_ts[...] += dts
    acc_sq[...] += dsq

    wsum = jnp.sum(xu8 * w3[i, :, :] + xt8 * w3[_NB + i, :, :]
                   + xb8 * w3[2 * _NB + i, :, :])
    acc_b[...] += jnp.reshape(wsum, (1, 1))

    @pl.when(i == _NB - 1)
    def _fin():
        u = acc_u[...]
        t = acc_ts[0:1, :]
        s_vec = acc_ts[1:2, :]
        u_t = jnp.sum(u * t)
        t_b = jnp.sum(t * s_vec)
        u_b = jnp.sum(u * s_vec)
        bs = 0.5 * (jnp.sum(s_vec * s_vec) - jnp.sum(acc_sq[...]))
        y = w0_ref[0, 0] + acc_b[0, 0] + u_t + t_b + bs + u_b
        out_ref[...] = jnp.reshape(y, (1, 1))


@jax.jit
def _fm(x, w_0, w_bias, u_V, b_V):
    x3 = x.reshape(3 * _NB, 8, _SUB)
    w3 = w_bias.reshape(3 * _NB, 8, _SUB)
    u4 = u_V.reshape(_NB, 8, _SUB, _K)
    b4 = b_V.reshape(_NB, 8, _SUB, _K)
    w0 = w_0.reshape(1, 1)
    vspec = pl.BlockSpec((1, 8, _SUB, _K), lambda i: (i, 0, 0, 0))
    return pl.pallas_call(
        _body,
        grid=(_NB,),
        in_specs=[
            pl.BlockSpec((1, 1), lambda i: (0, 0)),
            pl.BlockSpec((3 * _NB, 8, _SUB), lambda i: (0, 0, 0)),
            pl.BlockSpec((3 * _NB, 8, _SUB), lambda i: (0, 0, 0)),
            vspec, vspec,
        ],
        out_specs=pl.BlockSpec((1, 1), lambda i: (0, 0)),
        out_shape=jax.ShapeDtypeStruct((1, 1), jnp.float32),
        scratch_shapes=[
            pltpu.VMEM((1, _K), jnp.float32),
            pltpu.VMEM((2, _K), jnp.float32),
            pltpu.VMEM((1, _K), jnp.float32),
            pltpu.VMEM((1, 1), jnp.float32),
        ],
    )(w0, x3, w3, u4, b4)


def kernel(x, delta, pmi, w_0, w_bias, u_V, b_V):
    return _fm(x, w_0, w_bias, u_V, b_V)
